# baseline (speedup 1.0000x reference)
import functools

import jax
import jax.numpy as jnp
from jax import lax
from jax.experimental import pallas as pl
from jax.experimental.pallas import tpu as pltpu, tpu_sc as plsc

NC = 2
NS = 16
NW = NC * NS
CH = 128
DW = 16


def _sc_mesh():
  return plsc.VectorSubcoreMesh(
      core_axis_name="c", subcore_axis_name="s", num_cores=NC,
      num_subcores=NS)


def _make_deg_kernel(EP, NP, cpw):
  rpt = NP // NS

  @functools.partial(
      pl.kernel,
      mesh=_sc_mesh(),
      out_type=jax.ShapeDtypeStruct((NC * NP, DW), jnp.float32),
      scratch_types=[
          pltpu.VMEM((CH,), jnp.int32),
          pltpu.VMEM((CH, DW), jnp.float32),
          pltpu.VMEM_SHARED((NP, DW), jnp.float32),
      ],
  )
  def deg_kernel(dst_hbm, ones_hbm, zeros_hbm, degp_hbm, idx_v, ones_v,
                 acc_sh):
    c = lax.axis_index("c")
    s = lax.axis_index("s")
    wid = s * NC + c
    r0 = pl.multiple_of(s * rpt, 8)
    pltpu.sync_copy(zeros_hbm.at[pl.ds(r0, rpt)], acc_sh.at[pl.ds(r0, rpt)])
    pltpu.sync_copy(ones_hbm, ones_v)
    plsc.subcore_barrier()
    base = wid * (cpw * CH)

    def body(j, carry):
      st = pl.multiple_of(base + j * CH, CH)
      pltpu.sync_copy(dst_hbm.at[pl.ds(st, CH)], idx_v)
      pltpu.sync_copy(ones_v, acc_sh.at[idx_v], add=True)
      return carry

    lax.fori_loop(0, cpw, body, 0)
    plsc.subcore_barrier()
    o0 = pl.multiple_of(c * NP + s * rpt, 8)
    pltpu.sync_copy(acc_sh.at[pl.ds(r0, rpt)], degp_hbm.at[pl.ds(o0, rpt)])

  return deg_kernel


def _make_scatter_kernel(EP, NP, N, D, cpw):
  rpt = NP // NS
  assert cpw % 2 == 0

  @functools.partial(
      pl.kernel,
      mesh=_sc_mesh(),
      out_type=jax.ShapeDtypeStruct((NC * NP, D), jnp.float32),
      scratch_types=[
          pltpu.VMEM((cpw + 2, CH), jnp.int32),
          pltpu.VMEM((2, CH), jnp.int32),
          pltpu.VMEM((2, CH, D), jnp.float32),
          pltpu.VMEM_SHARED((NP, D), jnp.float32),
          pltpu.SemaphoreType.DMA,
          pltpu.SemaphoreType.DMA,
      ],
  )
  def scat_kernel(src_hbm, dst_hbm, g_hbm, zeros_hbm, zidx_hbm, part_hbm,
                  sidx_v, didx_v, rows_v, acc_sh, sem0, sem1):
    c = lax.axis_index("c")
    s = lax.axis_index("s")
    wid = s * NC + c
    r0 = pl.multiple_of(s * rpt, 8)
    pltpu.sync_copy(zeros_hbm.at[pl.ds(r0, rpt)], acc_sh.at[pl.ds(r0, rpt)])
    pltpu.sync_copy(src_hbm.at[wid], sidx_v.at[pl.ds(0, cpw)])
    pltpu.sync_copy(zidx_hbm, sidx_v.at[cpw])
    pltpu.sync_copy(zidx_hbm, sidx_v.at[cpw + 1])
    plsc.subcore_barrier()

    base = wid * (cpw * CH)
    sems = (sem0, sem1)

    def start(j, slot, with_didx):
      pltpu.async_copy(g_hbm.at[sidx_v.at[j]], rows_v.at[slot], sems[slot])
      if with_didx:
        st = pl.multiple_of(base + j * CH, CH)
        pltpu.async_copy(dst_hbm.at[pl.ds(st, CH)], didx_v.at[slot],
                         sems[slot])

    def wait(j, slot, with_didx):
      pltpu.make_async_copy(
          g_hbm.at[sidx_v.at[j]], rows_v.at[slot], sems[slot]).wait()
      if with_didx:
        st = pl.multiple_of(base + j * CH, CH)
        pltpu.make_async_copy(dst_hbm.at[pl.ds(st, CH)], didx_v.at[slot],
                              sems[slot]).wait()

    def scat(slot):
      pltpu.sync_copy(rows_v.at[slot], acc_sh.at[didx_v.at[slot]], add=True)

    start(0, 0, True)
    start(1, 1, True)

    def body(i, carry):
      j = 2 * i
      wait(j, 0, True)
      scat(0)
      start(j + 2, 0, True)
      wait(j + 1, 1, True)
      scat(1)
      start(j + 3, 1, True)
      return carry

    lax.fori_loop(0, cpw // 2 - 1, body, 0)
    j = cpw - 2
    wait(j, 0, True)
    scat(0)
    start(j + 2, 0, False)
    wait(j + 1, 1, True)
    scat(1)
    start(j + 3, 1, False)
    wait(cpw, 0, False)
    wait(cpw + 1, 1, False)
    plsc.subcore_barrier()
    o0 = pl.multiple_of(c * NP + s * rpt, 8)
    pltpu.sync_copy(acc_sh.at[pl.ds(r0, rpt)], part_hbm.at[pl.ds(o0, rpt)])

  return scat_kernel


def _mm_body(x_ref, w_ref, o_ref):
  o_ref[:] = jnp.dot(x_ref[:], w_ref[:], preferred_element_type=jnp.float32)


def _dis_body(dega_ref, degb_ref, p1_ref, dis_ref, g1_ref):
  d = dega_ref[:] + degb_ref[:] + 1.0
  dv = lax.rsqrt(d)
  dis_ref[:] = dv
  g1_ref[:] = p1_ref[:] * dv


def _mid_body(pa_ref, pb_ref, g1_ref, dis_ref, b1_ref, w2_ref, g2_ref):
  dv = dis_ref[:]
  h = (pa_ref[:] + pb_ref[:] + g1_ref[:]) * dv + b1_ref[:]
  h = jnp.maximum(h, 0.0)
  g2_ref[:] = jnp.dot(h, w2_ref[:], preferred_element_type=jnp.float32) * dv


def _fin_body(pa_ref, pb_ref, g2_ref, dis_ref, b2_ref, o_ref):
  o_ref[:] = (pa_ref[:] + pb_ref[:] + g2_ref[:]) * dis_ref[:] + b2_ref[:]


def kernel(x, edge_index, W1, b1, W2, b2):
  N, D = x.shape
  E = edge_index.shape[1]

  ei = edge_index.astype(jnp.int32)
  src, dst = ei[0], ei[1]

  NP = ((N + NS - 1) // NS) * NS
  if (NP // NS) % 8 != 0:
    NP = ((N + 8 * NS - 1) // (8 * NS)) * (8 * NS)
  cpw = (E + NW * CH - 1) // (NW * CH)
  cpw += cpw % 2
  EP = cpw * NW * CH
  if EP > E:
    pad = EP - E
    src = jnp.concatenate([src, jnp.zeros((pad,), jnp.int32)])
    dst = jnp.concatenate([dst, jnp.full((pad,), N, jnp.int32)])
  src3 = src.reshape(NW, cpw, CH)

  zeros_d = jnp.zeros((NP, D), jnp.float32)
  zeros_w = jnp.zeros((NP, DW), jnp.float32)
  ones_w = jnp.ones((CH, DW), jnp.float32)
  zidx = jnp.zeros((CH,), jnp.int32)

  deg_k = _make_deg_kernel(EP, NP, cpw)
  scat_k = _make_scatter_kernel(EP, NP, N, D, cpw)

  BR = 1000 if N % 1000 == 0 else 8
  grid = (pl.cdiv(N, BR),)
  row_spec = pl.BlockSpec((BR, D), lambda i: (i, 0))
  col_spec = pl.BlockSpec((BR, 1), lambda i: (i, 0))
  full_spec = pl.BlockSpec((D, D), lambda i: (0, 0))
  bias_spec = pl.BlockSpec((1, D), lambda i: (0, 0))

  p1 = pl.pallas_call(
      _mm_body,
      grid=grid,
      in_specs=[row_spec, full_spec],
      out_specs=row_spec,
      out_shape=jax.ShapeDtypeStruct((N, D), jnp.float32),
  )(x, W1)

  degp = deg_k(dst, ones_w, zeros_w)
  dega = degp[:N, :1]
  degb = degp[NP:NP + N, :1]

  dis, g1 = pl.pallas_call(
      _dis_body,
      grid=grid,
      in_specs=[col_spec, col_spec, row_spec],
      out_specs=[col_spec, row_spec],
      out_shape=[
          jax.ShapeDtypeStruct((N, 1), jnp.float32),
          jax.ShapeDtypeStruct((N, D), jnp.float32),
      ],
  )(dega, degb, p1)

  part1 = scat_k(src3, dst, g1, zeros_d, zidx)

  g2 = pl.pallas_call(
      _mid_body,
      grid=grid,
      in_specs=[row_spec, row_spec, row_spec, col_spec, bias_spec, full_spec],
      out_specs=row_spec,
      out_shape=jax.ShapeDtypeStruct((N, D), jnp.float32),
  )(part1[:N], part1[NP:NP + N], g1, dis, b1.reshape(1, D), W2)

  part2 = scat_k(src3, dst, g2, zeros_d, zidx)

  out = pl.pallas_call(
      _fin_body,
      grid=grid,
      in_specs=[row_spec, row_spec, row_spec, col_spec, bias_spec],
      out_specs=row_spec,
      out_shape=jax.ShapeDtypeStruct((N, D), jnp.float32),
  )(part2[:N], part2[NP:NP + N], g2, dis, b2.reshape(1, D))

  return out


# speedup vs baseline: 1.4129x; 1.4129x over previous
import functools

import jax
import jax.numpy as jnp
from jax import lax
from jax.experimental import pallas as pl
from jax.experimental.pallas import tpu as pltpu, tpu_sc as plsc

NC = 2
NS = 16
NW = NC * NS
CH = 128
DW = 16


def _sc_mesh():
  return plsc.VectorSubcoreMesh(
      core_axis_name="c", subcore_axis_name="s", num_cores=NC,
      num_subcores=NS)


def _make_deg_kernel(EP, NPD, cpw):
  rpt = NPD // NS
  assert rpt % CH == 0

  @functools.partial(
      pl.kernel,
      mesh=_sc_mesh(),
      out_type=jax.ShapeDtypeStruct((NC * NPD,), jnp.float32),
      scratch_types=[
          pltpu.VMEM((CH,), jnp.int32),
          pltpu.VMEM((CH,), jnp.float32),
          pltpu.VMEM((CH,), jnp.float32),
          pltpu.VMEM_SHARED((NPD,), jnp.float32),
      ],
  )
  def deg_kernel(dst_hbm, degp_hbm, idx_v, ones_v, zero_v, acc_sh):
    c = lax.axis_index("c")
    s = lax.axis_index("s")
    wid = s * NC + c
    r0 = pl.multiple_of(s * rpt, 8)
    for i in range(CH // 16):
      ones_v[pl.ds(16 * i, 16)] = jnp.full((16,), 1.0, jnp.float32)
      zero_v[pl.ds(16 * i, 16)] = jnp.zeros((16,), jnp.float32)

    def zbody(k, carry):
      pltpu.sync_copy(zero_v, acc_sh.at[pl.ds(r0 + k * CH, CH)])
      return carry

    lax.fori_loop(0, rpt // CH, zbody, 0)
    plsc.subcore_barrier()
    base = wid * (cpw * CH)

    def body(j, carry):
      st = pl.multiple_of(base + j * CH, CH)
      pltpu.sync_copy(dst_hbm.at[pl.ds(st, CH)], idx_v)
      pltpu.sync_copy(ones_v, acc_sh.at[idx_v], add=True)
      return carry

    lax.fori_loop(0, cpw, body, 0)
    plsc.subcore_barrier()
    o0 = pl.multiple_of(c * NPD + s * rpt, 8)
    pltpu.sync_copy(acc_sh.at[pl.ds(r0, rpt)], degp_hbm.at[pl.ds(o0, rpt)])

  return deg_kernel


def _make_scatter_kernel(EP, NP, N, D, cpw):
  rpt = NP // NS
  assert cpw % 2 == 0

  @functools.partial(
      pl.kernel,
      mesh=_sc_mesh(),
      out_type=jax.ShapeDtypeStruct((NC * NP, D), jnp.float32),
      scratch_types=[
          pltpu.VMEM((CH,), jnp.int32),
          pltpu.VMEM((CH,), jnp.int32),
          pltpu.VMEM((CH,), jnp.int32),
          pltpu.VMEM((CH,), jnp.int32),
          pltpu.VMEM((CH, D), jnp.float32),
          pltpu.VMEM((CH, D), jnp.float32),
          pltpu.VMEM_SHARED((NP, D), jnp.float32),
          pltpu.SemaphoreType.DMA,
          pltpu.SemaphoreType.DMA,
          pltpu.SemaphoreType.DMA,
          pltpu.SemaphoreType.DMA,
      ],
  )
  def scat_kernel(src_hbm, dst_hbm, g_hbm, zeros_hbm, part_hbm,
                  sidx0, sidx1, didx0, didx1, rows0, rows1, acc_sh,
                  semi0, semi1, semg0, semg1):
    c = lax.axis_index("c")
    s = lax.axis_index("s")
    wid = s * NC + c
    r0 = pl.multiple_of(s * rpt, 8)
    pltpu.sync_copy(zeros_hbm.at[pl.ds(r0, rpt)], acc_sh.at[pl.ds(r0, rpt)])
    plsc.subcore_barrier()

    base = wid * (cpw * CH)
    sidx = (sidx0, sidx1)
    didx = (didx0, didx1)
    rows = (rows0, rows1)
    semi = (semi0, semi1)
    semg = (semg0, semg1)

    def idx_load(j, slot):
      st = pl.multiple_of(base + j * CH, CH)
      d0 = pltpu.async_copy(src_hbm.at[pl.ds(st, CH)], sidx[slot],
                            semi[slot])
      d1 = pltpu.async_copy(dst_hbm.at[pl.ds(st, CH)], didx[slot],
                            semi[slot])
      return d0, d1

    def scat(slot):
      pltpu.sync_copy(rows[slot], acc_sh.at[didx[slot]], add=True)

    p0, p1 = idx_load(0, 0)
    p0.wait()
    p1.wait()

    def body(i, carry):
      j = 2 * i
      g0 = pltpu.async_copy(g_hbm.at[sidx[0]], rows[0], semg[0])
      g0.wait()
      scat(0)
      i1a, i1b = idx_load(j + 1, 1)
      i1a.wait()
      i1b.wait()
      g1 = pltpu.async_copy(g_hbm.at[sidx[1]], rows[1], semg[1])
      g1.wait()
      scat(1)
      i2a, i2b = idx_load(j + 2, 0)
      i2a.wait()
      i2b.wait()
      return carry

    lax.fori_loop(0, cpw // 2, body, 0)
    plsc.subcore_barrier()
    o0 = pl.multiple_of(c * NP + s * rpt, 8)
    pltpu.sync_copy(acc_sh.at[pl.ds(r0, rpt)], part_hbm.at[pl.ds(o0, rpt)])

  return scat_kernel


def _mm_body(x_ref, w_ref, o_ref):
  o_ref[:] = jnp.dot(x_ref[:], w_ref[:], preferred_element_type=jnp.float32)


def _dis_body(dega_ref, degb_ref, p1_ref, dis_ref, g1_ref):
  d = dega_ref[:] + degb_ref[:] + 1.0
  dv = lax.rsqrt(d)
  dis_ref[:] = dv
  g1_ref[:] = p1_ref[:] * dv


def _mid_body(pa_ref, pb_ref, g1_ref, dis_ref, b1_ref, w2_ref, g2_ref):
  dv = dis_ref[:]
  h = (pa_ref[:] + pb_ref[:] + g1_ref[:]) * dv + b1_ref[:]
  h = jnp.maximum(h, 0.0)
  g2_ref[:] = jnp.dot(h, w2_ref[:], preferred_element_type=jnp.float32) * dv


def _fin_body(pa_ref, pb_ref, g2_ref, dis_ref, b2_ref, o_ref):
  o_ref[:] = (pa_ref[:] + pb_ref[:] + g2_ref[:]) * dis_ref[:] + b2_ref[:]


def kernel(x, edge_index, W1, b1, W2, b2):
  N, D = x.shape
  E = edge_index.shape[1]

  ei = edge_index.astype(jnp.int32)
  src, dst = ei[0], ei[1]

  NP = ((N + NS - 1) // NS) * NS
  if (NP // NS) % 8 != 0:
    NP = ((N + 8 * NS - 1) // (8 * NS)) * (8 * NS)
  cpw = (E + NW * CH - 1) // (NW * CH)
  cpw += cpw % 2
  EP = cpw * NW * CH
  pad = EP + CH - E
  src = jnp.concatenate([src, jnp.zeros((pad,), jnp.int32)])
  dst = jnp.concatenate([dst, jnp.full((pad,), N, jnp.int32)])

  zeros_d = jnp.zeros((NP, D), jnp.float32)
  NPD = ((N + NS * CH - 1) // (NS * CH)) * (NS * CH)

  deg_k = _make_deg_kernel(EP, NPD, cpw)
  scat_k = _make_scatter_kernel(EP, NP, N, D, cpw)

  BR = 1000 if N % 1000 == 0 else 8
  grid = (pl.cdiv(N, BR),)
  row_spec = pl.BlockSpec((BR, D), lambda i: (i, 0))
  col_spec = pl.BlockSpec((BR, 1), lambda i: (i, 0))
  full_spec = pl.BlockSpec((D, D), lambda i: (0, 0))
  bias_spec = pl.BlockSpec((1, D), lambda i: (0, 0))

  p1 = pl.pallas_call(
      _mm_body,
      grid=grid,
      in_specs=[row_spec, full_spec],
      out_specs=row_spec,
      out_shape=jax.ShapeDtypeStruct((N, D), jnp.float32),
  )(x, W1)

  degp = deg_k(dst)
  dega = degp[:N].reshape(N, 1)
  degb = degp[NPD:NPD + N].reshape(N, 1)

  dis, g1 = pl.pallas_call(
      _dis_body,
      grid=grid,
      in_specs=[col_spec, col_spec, row_spec],
      out_specs=[col_spec, row_spec],
      out_shape=[
          jax.ShapeDtypeStruct((N, 1), jnp.float32),
          jax.ShapeDtypeStruct((N, D), jnp.float32),
      ],
  )(dega, degb, p1)

  part1 = scat_k(src, dst, g1, zeros_d)

  g2 = pl.pallas_call(
      _mid_body,
      grid=grid,
      in_specs=[row_spec, row_spec, row_spec, col_spec, bias_spec, full_spec],
      out_specs=row_spec,
      out_shape=jax.ShapeDtypeStruct((N, D), jnp.float32),
  )(part1[:N], part1[NP:NP + N], g1, dis, b1.reshape(1, D), W2)

  part2 = scat_k(src, dst, g2, zeros_d)

  out = pl.pallas_call(
      _fin_body,
      grid=grid,
      in_specs=[row_spec, row_spec, row_spec, col_spec, bias_spec],
      out_specs=row_spec,
      out_shape=jax.ShapeDtypeStruct((N, D), jnp.float32),
  )(part2[:N], part2[NP:NP + N], g2, dis, b2.reshape(1, D))

  return out


# speedup vs baseline: 1.5403x; 1.0902x over previous
import functools

import jax
import jax.numpy as jnp
from jax import lax
from jax.experimental import pallas as pl
from jax.experimental.pallas import tpu as pltpu, tpu_sc as plsc

NC = 2
NS = 16
NW = NC * NS
CH = 128
DW = 16


def _sc_mesh():
  return plsc.VectorSubcoreMesh(
      core_axis_name="c", subcore_axis_name="s", num_cores=NC,
      num_subcores=NS)


def _make_deg_kernel(EP, NPD, cpw):
  rpt = NPD // NS
  assert rpt % CH == 0

  @functools.partial(
      pl.kernel,
      mesh=_sc_mesh(),
      out_type=jax.ShapeDtypeStruct((NC * NPD,), jnp.float32),
      scratch_types=[
          pltpu.VMEM((CH,), jnp.int32),
          pltpu.VMEM((CH,), jnp.float32),
          pltpu.VMEM((CH,), jnp.float32),
          pltpu.VMEM_SHARED((NPD,), jnp.float32),
      ],
  )
  def deg_kernel(dst_hbm, degp_hbm, idx_v, ones_v, zero_v, acc_sh):
    c = lax.axis_index("c")
    s = lax.axis_index("s")
    wid = s * NC + c
    r0 = pl.multiple_of(s * rpt, 8)
    for i in range(CH // 16):
      ones_v[pl.ds(16 * i, 16)] = jnp.full((16,), 1.0, jnp.float32)
      zero_v[pl.ds(16 * i, 16)] = jnp.zeros((16,), jnp.float32)

    def zbody(k, carry):
      pltpu.sync_copy(zero_v, acc_sh.at[pl.ds(r0 + k * CH, CH)])
      return carry

    lax.fori_loop(0, rpt // CH, zbody, 0)
    plsc.subcore_barrier()
    base = wid * (cpw * CH)

    def body(j, carry):
      st = pl.multiple_of(base + j * CH, CH)
      pltpu.sync_copy(dst_hbm.at[pl.ds(st, CH)], idx_v)
      pltpu.sync_copy(ones_v, acc_sh.at[idx_v], add=True)
      return carry

    lax.fori_loop(0, cpw, body, 0)
    plsc.subcore_barrier()
    o0 = pl.multiple_of(c * NPD + s * rpt, 8)
    pltpu.sync_copy(acc_sh.at[pl.ds(r0, rpt)], degp_hbm.at[pl.ds(o0, rpt)])

  return deg_kernel


def _make_scatter_kernel(EP, NP, N, D, cpw):
  rpt = NP // NS
  assert cpw % 2 == 0

  @functools.partial(
      pl.kernel,
      mesh=_sc_mesh(),
      out_type=jax.ShapeDtypeStruct((NC * NP, D), jnp.float32),
      scratch_types=[
          pltpu.VMEM((CH,), jnp.int32),
          pltpu.VMEM((CH,), jnp.int32),
          pltpu.VMEM((CH,), jnp.int32),
          pltpu.VMEM((CH,), jnp.int32),
          pltpu.VMEM((CH, D), jnp.float32),
          pltpu.VMEM((CH, D), jnp.float32),
          pltpu.VMEM_SHARED((NP, D), jnp.float32),
          pltpu.SemaphoreType.DMA,
          pltpu.SemaphoreType.DMA,
          pltpu.SemaphoreType.DMA,
          pltpu.SemaphoreType.DMA,
      ],
  )
  def scat_kernel(src_hbm, dst_hbm, g_hbm, zeros_hbm, part_hbm,
                  sidx0, sidx1, didx0, didx1, rows0, rows1, acc_sh,
                  semi0, semi1, semg0, semg1):
    c = lax.axis_index("c")
    s = lax.axis_index("s")
    wid = s * NC + c
    r0 = pl.multiple_of(s * rpt, 8)
    pltpu.sync_copy(zeros_hbm.at[pl.ds(r0, rpt)], acc_sh.at[pl.ds(r0, rpt)])
    plsc.subcore_barrier()

    base = wid * (cpw * CH)
    sidx = (sidx0, sidx1)
    didx = (didx0, didx1)
    rows = (rows0, rows1)
    semi = (semi0, semi1)
    semg = (semg0, semg1)

    def idx_load(j, slot):
      st = pl.multiple_of(base + j * CH, CH)
      d0 = pltpu.async_copy(src_hbm.at[pl.ds(st, CH)], sidx[slot],
                            semi[slot])
      d1 = pltpu.async_copy(dst_hbm.at[pl.ds(st, CH)], didx[slot],
                            semi[slot])
      return d0, d1

    def scat(slot):
      pltpu.sync_copy(rows[slot], acc_sh.at[didx[slot]], add=True)

    p0, p1 = idx_load(0, 0)
    p0.wait()
    p1.wait()

    def body(i, carry):
      j = 2 * i
      g0 = pltpu.async_copy(g_hbm.at[sidx[0]], rows[0], semg[0])
      i1a, i1b = idx_load(j + 1, 1)
      i1a.wait()
      i1b.wait()
      g0.wait()
      g1 = pltpu.async_copy(g_hbm.at[sidx[1]], rows[1], semg[1])
      scat(0)
      i2a, i2b = idx_load(j + 2, 0)
      i2a.wait()
      i2b.wait()
      g1.wait()
      scat(1)
      return carry

    lax.fori_loop(0, cpw // 2, body, 0)
    plsc.subcore_barrier()
    o0 = pl.multiple_of(c * NP + s * rpt, 8)
    pltpu.sync_copy(acc_sh.at[pl.ds(r0, rpt)], part_hbm.at[pl.ds(o0, rpt)])

  return scat_kernel


def _mm_body(x_ref, w_ref, o_ref):
  o_ref[:] = jnp.dot(x_ref[:], w_ref[:], preferred_element_type=jnp.float32)


def _dis_body(dega_ref, degb_ref, p1_ref, dis_ref, g1_ref):
  d = dega_ref[:] + degb_ref[:] + 1.0
  dv = lax.rsqrt(d)
  dis_ref[:] = dv
  g1_ref[:] = p1_ref[:] * dv


def _mid_body(pa_ref, pb_ref, g1_ref, dis_ref, b1_ref, w2_ref, g2_ref):
  dv = dis_ref[:]
  h = (pa_ref[:] + pb_ref[:] + g1_ref[:]) * dv + b1_ref[:]
  h = jnp.maximum(h, 0.0)
  g2_ref[:] = jnp.dot(h, w2_ref[:], preferred_element_type=jnp.float32) * dv


def _fin_body(pa_ref, pb_ref, g2_ref, dis_ref, b2_ref, o_ref):
  o_ref[:] = (pa_ref[:] + pb_ref[:] + g2_ref[:]) * dis_ref[:] + b2_ref[:]


def kernel(x, edge_index, W1, b1, W2, b2):
  N, D = x.shape
  E = edge_index.shape[1]

  ei = edge_index.astype(jnp.int32)
  src, dst = ei[0], ei[1]

  NP = ((N + NS - 1) // NS) * NS
  if (NP // NS) % 8 != 0:
    NP = ((N + 8 * NS - 1) // (8 * NS)) * (8 * NS)
  cpw = (E + NW * CH - 1) // (NW * CH)
  cpw += cpw % 2
  EP = cpw * NW * CH
  pad = EP + CH - E
  src = jnp.concatenate([src, jnp.zeros((pad,), jnp.int32)])
  dst = jnp.concatenate([dst, jnp.full((pad,), N, jnp.int32)])

  zeros_d = jnp.zeros((NP, D), jnp.float32)
  NPD = ((N + NS * CH - 1) // (NS * CH)) * (NS * CH)

  deg_k = _make_deg_kernel(EP, NPD, cpw)
  scat_k = _make_scatter_kernel(EP, NP, N, D, cpw)

  BR = 1000 if N % 1000 == 0 else 8
  grid = (pl.cdiv(N, BR),)
  row_spec = pl.BlockSpec((BR, D), lambda i: (i, 0))
  col_spec = pl.BlockSpec((BR, 1), lambda i: (i, 0))
  full_spec = pl.BlockSpec((D, D), lambda i: (0, 0))
  bias_spec = pl.BlockSpec((1, D), lambda i: (0, 0))

  p1 = pl.pallas_call(
      _mm_body,
      grid=grid,
      in_specs=[row_spec, full_spec],
      out_specs=row_spec,
      out_shape=jax.ShapeDtypeStruct((N, D), jnp.float32),
  )(x, W1)

  degp = deg_k(dst)
  dega = degp[:N].reshape(N, 1)
  degb = degp[NPD:NPD + N].reshape(N, 1)

  dis, g1 = pl.pallas_call(
      _dis_body,
      grid=grid,
      in_specs=[col_spec, col_spec, row_spec],
      out_specs=[col_spec, row_spec],
      out_shape=[
          jax.ShapeDtypeStruct((N, 1), jnp.float32),
          jax.ShapeDtypeStruct((N, D), jnp.float32),
      ],
  )(dega, degb, p1)

  part1 = scat_k(src, dst, g1, zeros_d)

  g2 = pl.pallas_call(
      _mid_body,
      grid=grid,
      in_specs=[row_spec, row_spec, row_spec, col_spec, bias_spec, full_spec],
      out_specs=row_spec,
      out_shape=jax.ShapeDtypeStruct((N, D), jnp.float32),
  )(part1[:N], part1[NP:NP + N], g1, dis, b1.reshape(1, D), W2)

  part2 = scat_k(src, dst, g2, zeros_d)

  out = pl.pallas_call(
      _fin_body,
      grid=grid,
      in_specs=[row_spec, row_spec, row_spec, col_spec, bias_spec],
      out_specs=row_spec,
      out_shape=jax.ShapeDtypeStruct((N, D), jnp.float32),
  )(part2[:N], part2[NP:NP + N], g2, dis, b2.reshape(1, D))

  return out
